# baseline (speedup 1.0000x reference)
import functools

import jax
import jax.numpy as jnp
from jax.experimental import pallas as pl
from jax.experimental.pallas import tpu as pltpu

_NEGPOS_RATIO = 7.0
_OVERLAP = 0.35
_INV_VAR_S = 5.0
_MASKED = -1e9
_BISECT_ITERS = 42


def _loss_kernel(cls_ref, pbox_ref, tbox_ref, tgt_ref, priors_ref, out_ref,
                 bg_s, npos_s, rows_s, acc_s, *, n_batch, n_anchors,
                 n_objects):
    b = pl.program_id(0)
    f32 = jnp.float32
    eye4 = jnp.eye(4, dtype=f32)
    dn0 = (((0,), (1,)), ((), ()))
    dn1 = (((1,), (1,)), ((), ()))

    @pl.when(b == 0)
    def _prior_rows():
        prt = jax.lax.dot_general(eye4, priors_ref[...], dn1,
                                  preferred_element_type=f32)
        px0, py0, px1, py1 = (prt[0:1, :], prt[1:2, :], prt[2:3, :],
                              prt[3:4, :])
        pwx = jnp.clip(px1 - px0, 1e-6)
        pwy = jnp.clip(py1 - py0, 1e-6)
        invcx = 1.0 / (pwx * 0.1)
        invcy = 1.0 / (pwy * 0.1)
        rows_s[0:1, :] = invcx
        rows_s[1:2, :] = invcy
        rows_s[2:3, :] = (px0 + px1) * 0.5 * invcx
        rows_s[3:4, :] = (py0 + py1) * 0.5 * invcy
        rows_s[4:5, :] = jnp.log(pwx) * _INV_VAR_S
        rows_s[5:6, :] = jnp.log(pwy) * _INV_VAR_S
        rows_s[6:7, :] = (px1 - px0) * (py1 - py0)
        rows_s[8:12, :] = prt

    cls = cls_ref[0]
    tbox = tbox_ref[0]
    tgt = tgt_ref[0]
    pboxt = pbox_ref[0]
    px0, py0 = rows_s[8:9, :], rows_s[9:10, :]
    px1, py1 = rows_s[10:11, :], rows_s[11:12, :]

    tx0, ty0 = tbox[:, 0:1], tbox[:, 1:2]
    tx1, ty1 = tbox[:, 2:3], tbox[:, 3:4]
    iw = jnp.clip(jnp.minimum(tx1, px1) - jnp.maximum(tx0, px0), 0.0)
    ih = jnp.clip(jnp.minimum(ty1, py1) - jnp.maximum(ty0, py0), 0.0)
    inter = iw * ih
    area_t = (tx1 - tx0) * (ty1 - ty0)
    area_p = rows_s[6:7, :]
    iou = inter / jnp.clip(area_t + area_p - inter, 1e-9)

    lane = jax.lax.broadcasted_iota(jnp.int32, (n_objects, n_anchors), 1)
    iou_max = jnp.max(iou, axis=1, keepdims=True)
    best = jnp.min(jnp.where(iou == iou_max, lane, n_anchors), axis=1,
                   keepdims=True)
    posf = ((iou > _OVERLAP) | (lane == best)).astype(f32)
    pos_cnt = jnp.sum(posf, axis=0, keepdims=True)
    pos_any = pos_cnt > 0.0
    npos_b = jnp.sum(pos_any.astype(f32))
    pairs_b = jnp.sum(pos_cnt)

    cls0_col = cls[:, 0:1]
    e = jnp.exp(jnp.minimum(cls - cls0_col, 80.0))
    n_cls = cls.shape[1]
    ones_row = jnp.ones((1, n_cls), f32)
    sumexp = jax.lax.dot_general(ones_row, e, dn1, preferred_element_type=f32)
    bg = jnp.log(sumexp)
    class_iota = jax.lax.broadcasted_iota(jnp.int32, (n_cls, n_objects), 0)
    ohT = (class_iota == tgt).astype(f32)
    e0_row = (jax.lax.broadcasted_iota(jnp.int32, (1, n_cls), 1) == 0
              ).astype(f32)
    gathered = jax.lax.dot_general(ohT, cls, dn0,
                                   preferred_element_type=f32)
    cls0_row = jax.lax.dot_general(e0_row, cls, dn1,
                                   preferred_element_type=f32)
    lse_row = bg + cls0_row
    cep_b = jnp.sum(pos_cnt * lse_row) - jnp.sum(posf * gathered)

    bg_s[pl.ds(b, 1), :] = jnp.where(pos_any, _MASKED, bg)
    npos_s[pl.ds(b, 1), 0:1] = jnp.full((1, 1), npos_b, f32)

    invcx, invcy = rows_s[0:1, :], rows_s[1:2, :]
    qx = pboxt[0:1, :] + rows_s[2:3, :]
    qy = pboxt[1:2, :] + rows_s[3:4, :]
    qw = pboxt[2:3, :] + rows_s[4:5, :]
    qh = pboxt[3:4, :] + rows_s[5:6, :]
    tcx, tcy = (tx0 + tx1) * 0.5, (ty0 + ty1) * 0.5
    ltwx = jnp.log(jnp.clip(tx1 - tx0, 1e-6)) * _INV_VAR_S
    ltwy = jnp.log(jnp.clip(ty1 - ty0, 1e-6)) * _INV_VAR_S
    diffs = (
        qx - tcx * invcx,
        qy - tcy * invcy,
        qw - ltwx,
        qh - ltwy,
    )
    sl1sum = jnp.float32(0.0)
    for d in diffs:
        ad = jnp.abs(d)
        sl1sum = sl1sum + jnp.where(ad < 1.0, 0.5 * d * d, ad - 0.5)
    box_b = jnp.sum(sl1sum * posf)

    @pl.when(b == 0)
    def _init():
        acc_s[0] = pairs_b
        acc_s[1] = box_b
        acc_s[2] = cep_b

    @pl.when(b > 0)
    def _accum():
        acc_s[0] = acc_s[0] + pairs_b
        acc_s[1] = acc_s[1] + box_b
        acc_s[2] = acc_s[2] + cep_b

    @pl.when(b == n_batch - 1)
    def _finish():
        cand = bg_s[...]
        npos = npos_s[:, 0:1]
        k = jnp.minimum(_NEGPOS_RATIO * npos,
                        jnp.float32(n_anchors) - npos)
        unmasked = jnp.where(cand < -1e8, 0.0, cand)
        lo0 = jnp.min(unmasked, axis=1, keepdims=True) - 1.0
        hi0 = jnp.maximum(jnp.max(cand, axis=1, keepdims=True), lo0 + 1.0)

        def _bisect(_, lohi):
            lo, hi = lohi
            mid = 0.5 * (lo + hi)
            cnt = jnp.sum((cand > mid).astype(jnp.float32), axis=1,
                          keepdims=True)
            ge = cnt >= k
            return jnp.where(ge, mid, lo), jnp.where(ge, hi, mid)

        lo, _ = jax.lax.fori_loop(0, _BISECT_ITERS, _bisect, (lo0, hi0))
        above = cand > lo
        c = jnp.sum(above.astype(jnp.float32), axis=1, keepdims=True)
        s = jnp.sum(jnp.where(above, cand, 0.0), axis=1, keepdims=True)
        negsum = jnp.where(k > 0.5, s - (c - k) * lo, 0.0)

        pairs_tot = acc_s[0]
        n_sel = pairs_tot + jnp.sum(k)
        box_loss = acc_s[1] / jnp.maximum(pairs_tot, 1.0)
        cls_loss = (acc_s[2] + jnp.sum(negsum)) / jnp.maximum(n_sel, 1.0)
        out_ref[0] = box_loss
        out_ref[1] = cls_loss
        out_ref[2] = box_loss + cls_loss


def kernel(y_pred_boxes, y_pred_classes, y_true_boxes, priors, y_true_classes):
    B, A, C = y_pred_classes.shape
    O = y_true_boxes.shape[1]

    tgt3 = jnp.reshape(y_true_classes, (B, 1, O))
    pboxt = jnp.einsum("baj,jk->bka", y_pred_boxes,
                       jnp.eye(4, dtype=jnp.float32),
                       preferred_element_type=jnp.float32)

    grid = (B,)
    body = functools.partial(_loss_kernel, n_batch=B, n_anchors=A,
                             n_objects=O)
    return pl.pallas_call(
        body,
        grid=grid,
        in_specs=[
            pl.BlockSpec((1, A, C), lambda b: (b, 0, 0)),
            pl.BlockSpec((1, 4, A), lambda b: (b, 0, 0)),
            pl.BlockSpec((1, O, 4), lambda b: (b, 0, 0)),
            pl.BlockSpec((1, 1, O), lambda b: (b, 0, 0)),
            pl.BlockSpec((A, 4), lambda b: (0, 0)),
        ],
        out_specs=pl.BlockSpec(memory_space=pltpu.SMEM),
        out_shape=jax.ShapeDtypeStruct((3,), jnp.float32),
        scratch_shapes=[
            pltpu.VMEM((B, A), jnp.float32),
            pltpu.VMEM((B, 128), jnp.float32),
            pltpu.VMEM((12, A), jnp.float32),
            pltpu.SMEM((4,), jnp.float32),
        ],
        compiler_params=pltpu.CompilerParams(
            dimension_semantics=("arbitrary",),
        ),
    )(y_pred_classes, pboxt, y_true_boxes, tgt3, priors)


# speedup vs baseline: 1.1632x; 1.1632x over previous
import functools

import jax
import jax.numpy as jnp
from jax.experimental import pallas as pl
from jax.experimental.pallas import tpu as pltpu

_NEGPOS_RATIO = 7.0
_OVERLAP = 0.35
_INV_VAR_S = 5.0
_MASKED = -1e9
_BISECT_ITERS = 42


def _loss_kernel(clst_ref, pboxt_ref, tbox_ref, tgt_ref, priorst_ref,
                 out_ref, bg_s, npos_s, rows_s, acc_s, *, n_batch, n_anchors,
                 n_objects):
    b = pl.program_id(0)
    f32 = jnp.float32

    prt = priorst_ref[...]
    px0, py0, px1, py1 = prt[0:1, :], prt[1:2, :], prt[2:3, :], prt[3:4, :]

    @pl.when(b == 0)
    def _prior_rows():
        pwx = jnp.clip(px1 - px0, 1e-6)
        pwy = jnp.clip(py1 - py0, 1e-6)
        invcx = 1.0 / (pwx * 0.1)
        invcy = 1.0 / (pwy * 0.1)
        rows_s[0:1, :] = invcx
        rows_s[1:2, :] = invcy
        rows_s[2:3, :] = (px0 + px1) * 0.5 * invcx
        rows_s[3:4, :] = (py0 + py1) * 0.5 * invcy
        rows_s[4:5, :] = jnp.log(pwx) * _INV_VAR_S
        rows_s[5:6, :] = jnp.log(pwy) * _INV_VAR_S
        rows_s[6:7, :] = (px1 - px0) * (py1 - py0)

    clst = clst_ref[:, 0, 0, :]
    pboxt = pboxt_ref[0]
    tbox = tbox_ref[0]
    tgt = tgt_ref[0]

    tx0, ty0 = tbox[:, 0:1], tbox[:, 1:2]
    tx1, ty1 = tbox[:, 2:3], tbox[:, 3:4]
    iw = jnp.clip(jnp.minimum(tx1, px1) - jnp.maximum(tx0, px0), 0.0)
    ih = jnp.clip(jnp.minimum(ty1, py1) - jnp.maximum(ty0, py0), 0.0)
    inter = iw * ih
    area_t = (tx1 - tx0) * (ty1 - ty0)
    area_p = rows_s[6:7, :]
    iou = inter / jnp.clip(area_t + area_p - inter, 1e-9)

    lane = jax.lax.broadcasted_iota(jnp.int32, (n_objects, n_anchors), 1)
    iou_max = jnp.max(iou, axis=1, keepdims=True)
    best = jnp.min(jnp.where(iou == iou_max, lane, n_anchors), axis=1,
                   keepdims=True)
    posf = ((iou > _OVERLAP) | (lane == best)).astype(f32)
    pos_cnt = jnp.sum(posf, axis=0, keepdims=True)
    pos_any = pos_cnt > 0.0
    npos_b = jnp.sum(pos_any.astype(f32))
    pairs_b = jnp.sum(pos_cnt)

    cls0_row = clst[0:1, :]
    e = jnp.exp(jnp.minimum(clst - cls0_row, 80.0))
    sumexp = jnp.sum(e, axis=0, keepdims=True)
    bg = jnp.log(sumexp)
    n_cls = clst.shape[0]
    class_iota = jax.lax.broadcasted_iota(jnp.int32, (n_cls, n_objects), 0)
    ohT = (class_iota == tgt).astype(f32)
    gathered = jax.lax.dot_general(ohT, clst, (((0,), (0,)), ((), ())),
                                   preferred_element_type=f32)
    lse_row = bg + cls0_row
    cep_b = jnp.sum(pos_cnt * lse_row) - jnp.sum(posf * gathered)

    bg_s[pl.ds(b, 1), :] = jnp.where(pos_any, _MASKED, bg)
    npos_s[pl.ds(b, 1), 0:1] = jnp.full((1, 1), npos_b, f32)

    invcx, invcy = rows_s[0:1, :], rows_s[1:2, :]
    qx = pboxt[0:1, :] + rows_s[2:3, :]
    qy = pboxt[1:2, :] + rows_s[3:4, :]
    qw = pboxt[2:3, :] + rows_s[4:5, :]
    qh = pboxt[3:4, :] + rows_s[5:6, :]
    tcx, tcy = (tx0 + tx1) * 0.5, (ty0 + ty1) * 0.5
    ltwx = jnp.log(jnp.clip(tx1 - tx0, 1e-6)) * _INV_VAR_S
    ltwy = jnp.log(jnp.clip(ty1 - ty0, 1e-6)) * _INV_VAR_S
    diffs = (
        qx - tcx * invcx,
        qy - tcy * invcy,
        qw - ltwx,
        qh - ltwy,
    )
    sl1sum = jnp.float32(0.0)
    for d in diffs:
        ad = jnp.abs(d)
        sl1sum = sl1sum + jnp.where(ad < 1.0, 0.5 * d * d, ad - 0.5)
    box_b = jnp.sum(sl1sum * posf)

    @pl.when(b == 0)
    def _init():
        acc_s[0] = pairs_b
        acc_s[1] = box_b
        acc_s[2] = cep_b

    @pl.when(b > 0)
    def _accum():
        acc_s[0] = acc_s[0] + pairs_b
        acc_s[1] = acc_s[1] + box_b
        acc_s[2] = acc_s[2] + cep_b

    @pl.when(b == n_batch - 1)
    def _finish():
        cand = bg_s[...]
        npos = npos_s[:, 0:1]
        k = jnp.minimum(_NEGPOS_RATIO * npos,
                        jnp.float32(n_anchors) - npos)
        unmasked = jnp.where(cand < -1e8, 0.0, cand)
        lo0 = jnp.min(unmasked, axis=1, keepdims=True) - 1.0
        hi0 = jnp.maximum(jnp.max(cand, axis=1, keepdims=True), lo0 + 1.0)

        def _bisect(_, lohi):
            lo, hi = lohi
            mid = 0.5 * (lo + hi)
            cnt = jnp.sum((cand > mid).astype(jnp.float32), axis=1,
                          keepdims=True)
            ge = cnt >= k
            return jnp.where(ge, mid, lo), jnp.where(ge, hi, mid)

        lo, _ = jax.lax.fori_loop(0, _BISECT_ITERS, _bisect, (lo0, hi0))
        above = cand > lo
        c = jnp.sum(above.astype(jnp.float32), axis=1, keepdims=True)
        s = jnp.sum(jnp.where(above, cand, 0.0), axis=1, keepdims=True)
        negsum = jnp.where(k > 0.5, s - (c - k) * lo, 0.0)

        pairs_tot = acc_s[0]
        n_sel = pairs_tot + jnp.sum(k)
        box_loss = acc_s[1] / jnp.maximum(pairs_tot, 1.0)
        cls_loss = (acc_s[2] + jnp.sum(negsum)) / jnp.maximum(n_sel, 1.0)
        out_ref[0] = box_loss
        out_ref[1] = cls_loss
        out_ref[2] = box_loss + cls_loss


def kernel(y_pred_boxes, y_pred_classes, y_true_boxes, priors, y_true_classes):
    B, A, C = y_pred_classes.shape
    O = y_true_boxes.shape[1]

    clst = jnp.reshape(jnp.transpose(y_pred_classes, (2, 0, 1)),
                       (C, B, 1, A))
    pboxt = jnp.transpose(y_pred_boxes, (0, 2, 1))
    priorst = jnp.transpose(priors, (1, 0))
    tgt3 = jnp.reshape(y_true_classes, (B, 1, O))

    grid = (B,)
    body = functools.partial(_loss_kernel, n_batch=B, n_anchors=A,
                             n_objects=O)
    return pl.pallas_call(
        body,
        grid=grid,
        in_specs=[
            pl.BlockSpec((C, 1, 1, A), lambda b: (0, b, 0, 0)),
            pl.BlockSpec((1, 4, A), lambda b: (b, 0, 0)),
            pl.BlockSpec((1, O, 4), lambda b: (b, 0, 0)),
            pl.BlockSpec((1, 1, O), lambda b: (b, 0, 0)),
            pl.BlockSpec((4, A), lambda b: (0, 0)),
        ],
        out_specs=pl.BlockSpec(memory_space=pltpu.SMEM),
        out_shape=jax.ShapeDtypeStruct((3,), jnp.float32),
        scratch_shapes=[
            pltpu.VMEM((B, A), jnp.float32),
            pltpu.VMEM((B, 128), jnp.float32),
            pltpu.VMEM((8, A), jnp.float32),
            pltpu.SMEM((4,), jnp.float32),
        ],
        compiler_params=pltpu.CompilerParams(
            dimension_semantics=("arbitrary",),
        ),
    )(clst, pboxt, y_true_boxes, tgt3, priorst)


# speedup vs baseline: 1.4289x; 1.2284x over previous
import functools

import jax
import jax.numpy as jnp
from jax.experimental import pallas as pl
from jax.experimental.pallas import tpu as pltpu

_NEGPOS_RATIO = 7.0
_OVERLAP = 0.35
_INV_VAR_S = 5.0
_MASKED = -1e9
_BISECT_ITERS = 42


def _loss_kernel(clst_ref, pboxt_ref, tbox_ref, tgt_ref, priorst_ref,
                 out_ref, bg_s, npos_s, rows_s, acc_s, *, n_batch, n_anchors,
                 n_objects):
    b = pl.program_id(0)
    f32 = jnp.float32

    prt = priorst_ref[...]
    px0, py0, px1, py1 = prt[0:1, :], prt[1:2, :], prt[2:3, :], prt[3:4, :]

    @pl.when(b == 0)
    def _prior_rows():
        pwx = jnp.clip(px1 - px0, 1e-6)
        pwy = jnp.clip(py1 - py0, 1e-6)
        invcx = 1.0 / (pwx * 0.1)
        invcy = 1.0 / (pwy * 0.1)
        rows_s[0:1, :] = invcx
        rows_s[1:2, :] = invcy
        rows_s[2:3, :] = (px0 + px1) * 0.5 * invcx
        rows_s[3:4, :] = (py0 + py1) * 0.5 * invcy
        rows_s[4:5, :] = jnp.log(pwx) * _INV_VAR_S
        rows_s[5:6, :] = jnp.log(pwy) * _INV_VAR_S
        rows_s[6:7, :] = (px1 - px0) * (py1 - py0)

    clst = clst_ref[:, pl.ds(jax.lax.rem(b, 8), 1), :][:, 0, :]
    pboxt = pboxt_ref[0]
    tbox = tbox_ref[0]
    tgt = tgt_ref[0]

    tx0, ty0 = tbox[:, 0:1], tbox[:, 1:2]
    tx1, ty1 = tbox[:, 2:3], tbox[:, 3:4]
    iw = jnp.clip(jnp.minimum(tx1, px1) - jnp.maximum(tx0, px0), 0.0)
    ih = jnp.clip(jnp.minimum(ty1, py1) - jnp.maximum(ty0, py0), 0.0)
    inter = iw * ih
    area_t = (tx1 - tx0) * (ty1 - ty0)
    area_p = rows_s[6:7, :]
    iou = inter / jnp.clip(area_t + area_p - inter, 1e-9)

    lane = jax.lax.broadcasted_iota(jnp.int32, (n_objects, n_anchors), 1)
    iou_max = jnp.max(iou, axis=1, keepdims=True)
    best = jnp.min(jnp.where(iou == iou_max, lane, n_anchors), axis=1,
                   keepdims=True)
    posf = ((iou > _OVERLAP) | (lane == best)).astype(f32)
    pos_cnt = jnp.sum(posf, axis=0, keepdims=True)
    pos_any = pos_cnt > 0.0
    npos_b = jnp.sum(pos_any.astype(f32))
    pairs_b = jnp.sum(pos_cnt)

    cls0_row = clst[0:1, :]
    e = jnp.exp(jnp.minimum(clst - cls0_row, 80.0))
    sumexp = jnp.sum(e, axis=0, keepdims=True)
    bg = jnp.log(sumexp)
    n_cls = clst.shape[0]
    class_iota = jax.lax.broadcasted_iota(jnp.int32, (n_cls, n_objects), 0)
    ohT = (class_iota == tgt).astype(f32)
    gathered = jax.lax.dot_general(ohT, clst, (((0,), (0,)), ((), ())),
                                   preferred_element_type=f32)
    lse_row = bg + cls0_row
    cep_b = jnp.sum(pos_cnt * lse_row) - jnp.sum(posf * gathered)

    bg_s[pl.ds(b, 1), :] = jnp.where(pos_any, _MASKED, bg)
    npos_s[pl.ds(b, 1), 0:1] = jnp.full((1, 1), npos_b, f32)

    invcx, invcy = rows_s[0:1, :], rows_s[1:2, :]
    qx = pboxt[0:1, :] + rows_s[2:3, :]
    qy = pboxt[1:2, :] + rows_s[3:4, :]
    qw = pboxt[2:3, :] + rows_s[4:5, :]
    qh = pboxt[3:4, :] + rows_s[5:6, :]
    tcx, tcy = (tx0 + tx1) * 0.5, (ty0 + ty1) * 0.5
    ltwx = jnp.log(jnp.clip(tx1 - tx0, 1e-6)) * _INV_VAR_S
    ltwy = jnp.log(jnp.clip(ty1 - ty0, 1e-6)) * _INV_VAR_S
    diffs = (
        qx - tcx * invcx,
        qy - tcy * invcy,
        qw - ltwx,
        qh - ltwy,
    )
    sl1sum = jnp.float32(0.0)
    for d in diffs:
        ad = jnp.abs(d)
        sl1sum = sl1sum + jnp.where(ad < 1.0, 0.5 * d * d, ad - 0.5)
    box_b = jnp.sum(sl1sum * posf)

    @pl.when(b == 0)
    def _init():
        acc_s[0] = pairs_b
        acc_s[1] = box_b
        acc_s[2] = cep_b

    @pl.when(b > 0)
    def _accum():
        acc_s[0] = acc_s[0] + pairs_b
        acc_s[1] = acc_s[1] + box_b
        acc_s[2] = acc_s[2] + cep_b

    @pl.when(b == n_batch - 1)
    def _finish():
        cand = bg_s[...]
        npos = npos_s[:, 0:1]
        k = jnp.minimum(_NEGPOS_RATIO * npos,
                        jnp.float32(n_anchors) - npos)
        unmasked = jnp.where(cand < -1e8, 0.0, cand)
        lo0 = jnp.min(unmasked, axis=1, keepdims=True) - 1.0
        hi0 = jnp.maximum(jnp.max(cand, axis=1, keepdims=True), lo0 + 1.0)

        def _bisect(_, lohi):
            lo, hi = lohi
            mid = 0.5 * (lo + hi)
            cnt = jnp.sum((cand > mid).astype(jnp.float32), axis=1,
                          keepdims=True)
            ge = cnt >= k
            return jnp.where(ge, mid, lo), jnp.where(ge, hi, mid)

        lo, _ = jax.lax.fori_loop(0, _BISECT_ITERS, _bisect, (lo0, hi0))
        above = cand > lo
        c = jnp.sum(above.astype(jnp.float32), axis=1, keepdims=True)
        s = jnp.sum(jnp.where(above, cand, 0.0), axis=1, keepdims=True)
        negsum = jnp.where(k > 0.5, s - (c - k) * lo, 0.0)

        pairs_tot = acc_s[0]
        n_sel = pairs_tot + jnp.sum(k)
        box_loss = acc_s[1] / jnp.maximum(pairs_tot, 1.0)
        cls_loss = (acc_s[2] + jnp.sum(negsum)) / jnp.maximum(n_sel, 1.0)
        out_ref[0] = box_loss
        out_ref[1] = cls_loss
        out_ref[2] = box_loss + cls_loss


def kernel(y_pred_boxes, y_pred_classes, y_true_boxes, priors, y_true_classes):
    B, A, C = y_pred_classes.shape
    O = y_true_boxes.shape[1]

    clst = jnp.transpose(y_pred_classes, (2, 0, 1))
    pboxt = jnp.transpose(y_pred_boxes, (0, 2, 1))
    priorst = jnp.transpose(priors, (1, 0))
    tgt3 = jnp.reshape(y_true_classes, (B, 1, O))

    grid = (B,)
    body = functools.partial(_loss_kernel, n_batch=B, n_anchors=A,
                             n_objects=O)
    return pl.pallas_call(
        body,
        grid=grid,
        in_specs=[
            pl.BlockSpec((C, 8, A), lambda b: (0, b // 8, 0)),
            pl.BlockSpec((1, 4, A), lambda b: (b, 0, 0)),
            pl.BlockSpec((1, O, 4), lambda b: (b, 0, 0)),
            pl.BlockSpec((1, 1, O), lambda b: (b, 0, 0)),
            pl.BlockSpec((4, A), lambda b: (0, 0)),
        ],
        out_specs=pl.BlockSpec(memory_space=pltpu.SMEM),
        out_shape=jax.ShapeDtypeStruct((3,), jnp.float32),
        scratch_shapes=[
            pltpu.VMEM((B, A), jnp.float32),
            pltpu.VMEM((B, 128), jnp.float32),
            pltpu.VMEM((8, A), jnp.float32),
            pltpu.SMEM((4,), jnp.float32),
        ],
        compiler_params=pltpu.CompilerParams(
            dimension_semantics=("arbitrary",),
        ),
    )(clst, pboxt, y_true_boxes, tgt3, priorst)
